# baseline (speedup 1.0000x reference)
import functools

import jax
import jax.numpy as jnp
from jax import lax
from jax.experimental import pallas as pl
from jax.experimental.pallas import tpu as pltpu
from jax.experimental.pallas import tpu_sc as plsc

_EPS = 1e-5

_N = 50000
_K = 16
_PTS_PER_CHUNK = 8
_NCHUNK = _N // _PTS_PER_CHUNK
_NWORKERS = 32
_CHUNKS_PER_W = 200
_IDX_ROWS_PER_W = _CHUNKS_PER_W
_IDX_ROWS = _NWORKERS * _IDX_ROWS_PER_W
_NBUF = 5
_NB = 8192


def _sc_gather_pool(T, idx2d, cvec):
    mesh = plsc.VectorSubcoreMesh(core_axis_name="c", subcore_axis_name="s")
    P = _PTS_PER_CHUNK
    R = P * _K
    CH = _CHUNKS_PER_W

    @functools.partial(
        pl.kernel,
        mesh=mesh,
        out_type=jax.ShapeDtypeStruct((_N, 32), jnp.uint32),
        scratch_types=[
            pltpu.VMEM((_IDX_ROWS_PER_W * 128,), jnp.int32),
        ]
        + [pltpu.VMEM((R, 128), jnp.uint32) for _ in range(_NBUF)]
        + [pltpu.VMEM((P, 128), jnp.uint32) for _ in range(_NBUF)]
        + [pltpu.VMEM((P, 32), jnp.uint32) for _ in range(_NBUF)]
        + [pltpu.VMEM((1, 32), jnp.uint32)]
        + [pltpu.SemaphoreType.DMA for _ in range(2 * _NBUF)],
    )
    def k(t_hbm, idx_hbm, c_hbm, out_hbm, idxw, *bufs):
        nt = bufs[0:_NBUF]
        ot = bufs[_NBUF:2 * _NBUF]
        og = bufs[2 * _NBUF:3 * _NBUF]
        cv = bufs[3 * _NBUF]
        sg = bufs[3 * _NBUF + 1:3 * _NBUF + 1 + _NBUF]
        so = bufs[3 * _NBUF + 1 + _NBUF:]
        wid = lax.axis_index("s") * 2 + lax.axis_index("c")
        lo = wid * CH
        nch = jnp.minimum(lo + CH, _NCHUNK) - lo
        pltpu.sync_copy(c_hbm, cv)
        pltpu.sync_copy(
            idx_hbm.at[pl.ds(wid * (_IDX_ROWS_PER_W * 128), _IDX_ROWS_PER_W * 128)],
            idxw)

        def issue(i, b):
            li = jnp.minimum(i, nch - 1)
            base = (lo + li) * P
            pltpu.async_copy(t_hbm.at[idxw.at[pl.ds(li * 128, 128)]], nt[b], sg[b])
            pltpu.async_copy(t_hbm.at[pl.ds(base, P)], ot[b], sg[b])

        def wait_gathers(b):
            pltpu.make_async_copy(t_hbm.at[pl.ds(0, R)], nt[b], sg[b]).wait()
            pltpu.make_async_copy(t_hbm.at[pl.ds(0, P)], ot[b], sg[b]).wait()

        def wait_out(b):
            pltpu.make_async_copy(out_hbm.at[pl.ds(0, P)], og[b], so[b]).wait()

        for j in range(_NBUF - 1):
            issue(j, j)

        def outer(ii, carry):
            for b in range(_NBUF):
                i = _NBUF * ii + b
                wait_gathers(b)

                @pl.when(i + _NBUF - 1 < CH)
                def _():
                    issue(i + _NBUF - 1, (b + _NBUF - 1) % _NBUF)

                @pl.when(ii >= 1)
                def _():
                    wait_out(b)

                ntv = nt[b].bitcast(jnp.bfloat16)
                otv = ot[b].bitcast(jnp.bfloat16)
                ogv = og[b].bitcast(jnp.bfloat16)
                cvv = cv.bitcast(jnp.bfloat16)

                cv0 = cvv[pl.ds(0, 2), pl.ds(0, 16)]
                cv1 = cvv[pl.ds(0, 2), pl.ds(16, 16)]

                def point_body(p, carry2):
                    p2 = pl.multiple_of(2 * p, 2)
                    vn0 = otv[pl.ds(p2, 2), pl.ds(0, 16)] + cv0
                    vn1 = otv[pl.ds(p2, 2), pl.ds(16, 16)] + cv1
                    z = jnp.zeros((2, 16), jnp.bfloat16)
                    a0, a1, b0, b1 = z, z, z, z
                    r0 = p * _K
                    for kk in range(_K):
                        r2 = pl.multiple_of(2 * (r0 + kk), 2)
                        t0 = (jnp.maximum(vn0 - ntv[pl.ds(r2, 2), pl.ds(0, 16)], 0.0)
                              * ntv[pl.ds(r2, 2), pl.ds(32, 16)])
                        t1 = (jnp.maximum(vn1 - ntv[pl.ds(r2, 2), pl.ds(16, 16)], 0.0)
                              * ntv[pl.ds(r2, 2), pl.ds(48, 16)])
                        if kk % 2 == 0:
                            a0 = a0 + t0
                            a1 = a1 + t1
                        else:
                            b0 = b0 + t0
                            b1 = b1 + t1
                    ogv[pl.ds(p2, 2), pl.ds(0, 16)] = a0 + b0
                    ogv[pl.ds(p2, 2), pl.ds(16, 16)] = a1 + b1
                    return carry2

                lax.fori_loop(0, P, point_body, 0)

                li = jnp.minimum(i, nch - 1)
                pltpu.async_copy(og[b], out_hbm.at[pl.ds((lo + li) * P, P)], so[b])
            return carry

        lax.fori_loop(0, CH // _NBUF, outer, 0)
        for b in range(_NBUF):
            wait_out(b)

    return k(T, idx2d, cvec)


def _full(shape):
    return pl.BlockSpec(shape, lambda i: tuple(0 for _ in shape))


def _pack2(lo, hi):
    l16 = lax.bitcast_convert_type(lo.astype(jnp.bfloat16), jnp.uint16)
    h16 = lax.bitcast_convert_type(hi.astype(jnp.bfloat16), jnp.uint16)
    return l16.astype(jnp.uint32) | (h16.astype(jnp.uint32) << 16)


def _unpack2(raw):
    lo = lax.bitcast_convert_type(raw << 16, jnp.float32)
    hi = lax.bitcast_convert_type(raw & jnp.uint32(0xFFFF0000), jnp.float32)
    return jnp.concatenate([lo, hi], axis=1)


def _stage_a(feat2d, xyz, Wm1, cm1, Wa1, Wa2):
    n_blocks = pl.cdiv(_N, _NB)

    def body(feat_ref, xyz_ref, wm1_ref, cm1_ref,
             wa1_ref, wa2_ref, t1_ref, v2_ref):
        X = feat_ref[...].astype(jnp.bfloat16)
        f1 = lax.dot_general(X, wm1_ref[...], (((0,), (1,)), ((), ())),
                             preferred_element_type=jnp.float32)
        f1 = jnp.maximum(f1 + cm1_ref[...], 0.0)
        xb = xyz_ref[0]
        wa1 = wa1_ref[...]
        wa2 = wa2_ref[...]
        v1 = (xb[:, 0:1] * wa1[0:1, :]
              + xb[:, 1:2] * wa1[1:2, :]
              + xb[:, 2:3] * wa1[2:3, :])
        v2 = (xb[:, 0:1] * wa2[0:1, :]
              + xb[:, 1:2] * wa2[1:2, :]
              + xb[:, 2:3] * wa2[2:3, :])
        t1_ref[:, 0:32] = _pack2(v1[:, 0:32], v1[:, 32:64])
        t1_ref[:, 32:64] = _pack2(f1[:, 0:32], f1[:, 32:64])
        t1_ref[:, 64:128] = jnp.zeros((t1_ref.shape[0], 64), jnp.uint32)
        v2_ref[...] = v2

    return pl.pallas_call(
        body,
        grid=(n_blocks,),
        in_specs=[
            pl.BlockSpec((128, _NB), lambda i: (0, i)),
            pl.BlockSpec((1, _NB, 3), lambda i: (0, i, 0)),
            _full((64, 128)), _full((1, 64)),
            _full((3, 64)), _full((3, 64)),
        ],
        out_specs=[
            pl.BlockSpec((_NB, 128), lambda i: (i, 0)),
            pl.BlockSpec((_NB, 64), lambda i: (i, 0)),
        ],
        out_shape=[
            jax.ShapeDtypeStruct((_N, 128), jnp.uint32),
            jax.ShapeDtypeStruct((_N, 64), jnp.float32),
        ],
    )(feat2d, xyz, Wm1, cm1, Wa1, Wa2)


def _stage_shortcut(feat2d, Wsc, csc):
    n_blocks = pl.cdiv(_N, _NB)

    def body(feat_ref, wsc_ref, csc_ref, sco_ref):
        X = feat_ref[...].astype(jnp.bfloat16)
        sco = lax.dot_general(wsc_ref[...], X, (((1,), (0,)), ((), ())),
                              preferred_element_type=jnp.float32)
        sco_ref[...] = jnp.maximum(sco + csc_ref[...], 0.0).astype(jnp.bfloat16)

    return pl.pallas_call(
        body,
        grid=(n_blocks,),
        in_specs=[
            pl.BlockSpec((128, _NB), lambda i: (0, i)),
            _full((256, 128)), _full((256, 1)),
        ],
        out_specs=pl.BlockSpec((256, _NB), lambda i: (0, i)),
        out_shape=jax.ShapeDtypeStruct((256, _N), jnp.bfloat16),
    )(feat2d, Wsc, csc)


def _stage_mid(G, V2, W, c):
    n_blocks = pl.cdiv(_N, _NB)

    def body(g_ref, v2_ref, w_ref, c_ref, t2_ref):
        g = _unpack2(g_ref[...]).astype(jnp.bfloat16)
        y = lax.dot_general(g, w_ref[...], (((1,), (1,)), ((), ())),
                            preferred_element_type=jnp.float32)
        f2 = jnp.maximum(y + c_ref[...], 0.0)
        v2 = v2_ref[...]
        t2_ref[:, 0:32] = _pack2(v2[:, 0:32], v2[:, 32:64])
        t2_ref[:, 32:64] = _pack2(f2[:, 0:32], f2[:, 32:64])
        t2_ref[:, 64:128] = jnp.zeros((t2_ref.shape[0], 64), jnp.uint32)

    return pl.pallas_call(
        body,
        grid=(n_blocks,),
        in_specs=[pl.BlockSpec((_NB, 32), lambda i: (i, 0)),
                  pl.BlockSpec((_NB, 64), lambda i: (i, 0)),
                  _full((64, 64)), _full((1, 64))],
        out_specs=pl.BlockSpec((_NB, 128), lambda i: (i, 0)),
        out_shape=jax.ShapeDtypeStruct((_N, 128), jnp.uint32),
    )(G, V2, W, c)


def _stage_out(G2, SCo, Wb2b, cb2b, Wm2, cm2):
    n_blocks = pl.cdiv(_N, _NB)

    def body(g_ref, sco_ref, wb_ref, cb_ref, wm_ref, cm_ref, o_ref):
        g = _unpack2(g_ref[...]).astype(jnp.bfloat16)
        f3 = lax.dot_general(g, wb_ref[...], (((1,), (1,)), ((), ())),
                             preferred_element_type=jnp.float32)
        f3 = jnp.maximum(f3 + cb_ref[...], 0.0).astype(jnp.bfloat16)
        f4 = lax.dot_general(wm_ref[...], f3, (((1,), (1,)), ((), ())),
                             preferred_element_type=jnp.float32)
        f4 = jnp.maximum(f4 + cm_ref[...], 0.0)
        y = f4 + sco_ref[...].astype(jnp.float32)
        o_ref[...] = jnp.maximum(y, 0.2 * y)

    return pl.pallas_call(
        body,
        grid=(n_blocks,),
        in_specs=[pl.BlockSpec((_NB, 32), lambda i: (i, 0)),
                  pl.BlockSpec((256, _NB), lambda i: (0, i)),
                  _full((128, 64)), _full((1, 128)),
                  _full((256, 128)), _full((256, 1))],
        out_specs=pl.BlockSpec((256, _NB), lambda i: (0, i)),
        out_shape=jax.ShapeDtypeStruct((256, _N), jnp.float32),
    )(G2, SCo, Wb2b, cb2b, Wm2, cm2)


def kernel(feature, xyz, neigh_idx,
           W_m1, b_m1, g_m1, be_m1,
           W_b1a, b_b1a, g_b1a, be_b1a,
           W_b1b, b_b1b, g_b1b, be_b1b,
           W_b2a, b_b2a, g_b2a, be_b2a,
           W_b2b, b_b2b, g_b2b, be_b2b,
           W_m2, b_m2, g_m2, be_m2,
           W_sc, b_sc, g_sc, be_sc):
    inv = 1.0 / jnp.sqrt(1.0 + _EPS)

    def scale(W, b, g, be):
        s = g * inv
        return W * s[:, None], (b * s + be)

    We_m1, ce_m1 = scale(W_m1, b_m1, g_m1, be_m1)
    We_b1a, ce_b1a = scale(W_b1a, b_b1a, g_b1a, be_b1a)
    We_b1b, ce_b1b = scale(W_b1b, b_b1b, g_b1b, be_b1b)
    We_b2a, ce_b2a = scale(W_b2a, b_b2a, g_b2a, be_b2a)
    We_b2b, ce_b2b = scale(W_b2b, b_b2b, g_b2b, be_b2b)
    We_m2, ce_m2 = scale(W_m2, b_m2, g_m2, be_m2)
    We_sc, ce_sc = scale(W_sc, b_sc, g_sc, be_sc)

    bf = jnp.bfloat16
    feat2d = feature[0, :, :, 0]
    idxflat = neigh_idx[0].reshape(_N * _K).astype(jnp.int32)
    idx2d = jnp.pad(idxflat, (0, _IDX_ROWS * 128 - _N * _K))

    T1, V2 = _stage_a(
        feat2d, xyz,
        We_m1.astype(bf), ce_m1.reshape(1, 64),
        jnp.transpose(We_b1a), jnp.transpose(We_b2a))
    SCo = _stage_shortcut(feat2d, We_sc.astype(bf), ce_sc.reshape(256, 1))

    G1 = _sc_gather_pool(T1, idx2d, _pack2(ce_b1a[:32], ce_b1a[32:]).reshape(1, 32))
    T2 = _stage_mid(G1, V2, (We_b1b * (1.0 / _K)).astype(bf), ce_b1b.reshape(1, 64))

    G2 = _sc_gather_pool(T2, idx2d, _pack2(ce_b2a[:32], ce_b2a[32:]).reshape(1, 32))
    out = _stage_out(G2, SCo, (We_b2b * (1.0 / _K)).astype(bf), ce_b2b.reshape(1, 128),
                     We_m2.astype(bf), ce_m2.reshape(256, 1))

    return out.reshape(1, 256, _N, 1)


# speedup vs baseline: 1.0132x; 1.0132x over previous
import functools

import jax
import jax.numpy as jnp
from jax import lax
from jax.experimental import pallas as pl
from jax.experimental.pallas import tpu as pltpu
from jax.experimental.pallas import tpu_sc as plsc

_EPS = 1e-5

_N = 50000
_K = 16
_PTS_PER_CHUNK = 8
_NCHUNK = _N // _PTS_PER_CHUNK
_NWORKERS = 32
_CHUNKS_PER_W = 200
_IDX_ROWS_PER_W = _CHUNKS_PER_W
_IDX_ROWS = _NWORKERS * _IDX_ROWS_PER_W
_NBUF = 4
_NB = 8192


def _sc_gather_pool(T, idx2d, cvec):
    mesh = plsc.VectorSubcoreMesh(core_axis_name="c", subcore_axis_name="s")
    P = _PTS_PER_CHUNK
    R = P * _K
    CH = _CHUNKS_PER_W

    @functools.partial(
        pl.kernel,
        mesh=mesh,
        out_type=jax.ShapeDtypeStruct((_N, 32), jnp.uint32),
        scratch_types=[
            pltpu.VMEM((_IDX_ROWS_PER_W * 128,), jnp.int32),
        ]
        + [pltpu.VMEM((R, 128), jnp.uint32) for _ in range(_NBUF)]
        + [pltpu.VMEM((P, 128), jnp.uint32) for _ in range(_NBUF)]
        + [pltpu.VMEM((P, 32), jnp.uint32) for _ in range(_NBUF)]
        + [pltpu.VMEM((1, 32), jnp.uint32)]
        + [pltpu.SemaphoreType.DMA for _ in range(2 * _NBUF)],
    )
    def k(t_hbm, idx_hbm, c_hbm, out_hbm, idxw, *bufs):
        nt = bufs[0:_NBUF]
        ot = bufs[_NBUF:2 * _NBUF]
        og = bufs[2 * _NBUF:3 * _NBUF]
        cv = bufs[3 * _NBUF]
        sg = bufs[3 * _NBUF + 1:3 * _NBUF + 1 + _NBUF]
        so = bufs[3 * _NBUF + 1 + _NBUF:]
        wid = lax.axis_index("s") * 2 + lax.axis_index("c")
        lo = wid * CH
        nch = jnp.minimum(lo + CH, _NCHUNK) - lo
        pltpu.sync_copy(c_hbm, cv)
        pltpu.sync_copy(
            idx_hbm.at[pl.ds(wid * (_IDX_ROWS_PER_W * 128), _IDX_ROWS_PER_W * 128)],
            idxw)

        def issue(i, b):
            li = jnp.minimum(i, nch - 1)
            base = (lo + li) * P
            pltpu.async_copy(t_hbm.at[idxw.at[pl.ds(li * 128, 128)]], nt[b], sg[b])
            pltpu.async_copy(t_hbm.at[pl.ds(base, P)], ot[b], sg[b])

        def wait_gathers(b):
            pltpu.make_async_copy(t_hbm.at[pl.ds(0, R)], nt[b], sg[b]).wait()
            pltpu.make_async_copy(t_hbm.at[pl.ds(0, P)], ot[b], sg[b]).wait()

        def wait_out(b):
            pltpu.make_async_copy(out_hbm.at[pl.ds(0, P)], og[b], so[b]).wait()

        for j in range(_NBUF - 1):
            issue(j, j)

        def outer(ii, carry):
            for b in range(_NBUF):
                i = _NBUF * ii + b
                wait_gathers(b)

                @pl.when(i + _NBUF - 1 < CH)
                def _():
                    issue(i + _NBUF - 1, (b + _NBUF - 1) % _NBUF)

                @pl.when(ii >= 1)
                def _():
                    wait_out(b)

                ntv = nt[b].bitcast(jnp.bfloat16)
                otv = ot[b].bitcast(jnp.bfloat16)
                ogv = og[b].bitcast(jnp.bfloat16)
                cvv = cv.bitcast(jnp.bfloat16)

                cv0 = cvv[pl.ds(0, 2), pl.ds(0, 16)]
                cv1 = cvv[pl.ds(0, 2), pl.ds(16, 16)]

                def point_body(p, carry2):
                    p2 = pl.multiple_of(2 * p, 2)
                    vn0 = otv[pl.ds(p2, 2), pl.ds(0, 16)] + cv0
                    vn1 = otv[pl.ds(p2, 2), pl.ds(16, 16)] + cv1
                    z = jnp.zeros((2, 16), jnp.bfloat16)
                    a0, a1, b0, b1 = z, z, z, z
                    r0 = p * _K
                    for kk in range(_K):
                        r2 = pl.multiple_of(2 * (r0 + kk), 2)
                        t0 = (jnp.maximum(vn0 - ntv[pl.ds(r2, 2), pl.ds(0, 16)], 0.0)
                              * ntv[pl.ds(r2, 2), pl.ds(32, 16)])
                        t1 = (jnp.maximum(vn1 - ntv[pl.ds(r2, 2), pl.ds(16, 16)], 0.0)
                              * ntv[pl.ds(r2, 2), pl.ds(48, 16)])
                        if kk % 2 == 0:
                            a0 = a0 + t0
                            a1 = a1 + t1
                        else:
                            b0 = b0 + t0
                            b1 = b1 + t1
                    ogv[pl.ds(p2, 2), pl.ds(0, 16)] = a0 + b0
                    ogv[pl.ds(p2, 2), pl.ds(16, 16)] = a1 + b1
                    return carry2

                lax.fori_loop(0, P, point_body, 0)

                li = jnp.minimum(i, nch - 1)
                pltpu.async_copy(og[b], out_hbm.at[pl.ds((lo + li) * P, P)], so[b])
            return carry

        lax.fori_loop(0, CH // _NBUF, outer, 0)
        for b in range(_NBUF):
            wait_out(b)

    return k(T, idx2d, cvec)


def _full(shape):
    return pl.BlockSpec(shape, lambda i: tuple(0 for _ in shape))


def _pack2(lo, hi):
    l16 = lax.bitcast_convert_type(lo.astype(jnp.bfloat16), jnp.uint16)
    h16 = lax.bitcast_convert_type(hi.astype(jnp.bfloat16), jnp.uint16)
    return l16.astype(jnp.uint32) | (h16.astype(jnp.uint32) << 16)


def _unpack2(raw):
    lo = lax.bitcast_convert_type(raw << 16, jnp.float32)
    hi = lax.bitcast_convert_type(raw & jnp.uint32(0xFFFF0000), jnp.float32)
    return jnp.concatenate([lo, hi], axis=1)


def _stage_a(feat2d, xyz, Wm1, cm1, Wa1, Wa2):
    n_blocks = pl.cdiv(_N, _NB)

    def body(feat_ref, xyz_ref, wm1_ref, cm1_ref,
             wa1_ref, wa2_ref, t1_ref, v2_ref):
        X = feat_ref[...].astype(jnp.bfloat16)
        f1 = lax.dot_general(X, wm1_ref[...], (((0,), (1,)), ((), ())),
                             preferred_element_type=jnp.float32)
        f1 = jnp.maximum(f1 + cm1_ref[...], 0.0)
        xb = xyz_ref[0]
        wa1 = wa1_ref[...]
        wa2 = wa2_ref[...]
        v1 = (xb[:, 0:1] * wa1[0:1, :]
              + xb[:, 1:2] * wa1[1:2, :]
              + xb[:, 2:3] * wa1[2:3, :])
        v2 = (xb[:, 0:1] * wa2[0:1, :]
              + xb[:, 1:2] * wa2[1:2, :]
              + xb[:, 2:3] * wa2[2:3, :])
        t1_ref[:, 0:32] = _pack2(v1[:, 0:32], v1[:, 32:64])
        t1_ref[:, 32:64] = _pack2(f1[:, 0:32], f1[:, 32:64])
        t1_ref[:, 64:128] = jnp.zeros((t1_ref.shape[0], 64), jnp.uint32)
        v2_ref[...] = v2

    return pl.pallas_call(
        body,
        grid=(n_blocks,),
        in_specs=[
            pl.BlockSpec((128, _NB), lambda i: (0, i)),
            pl.BlockSpec((1, _NB, 3), lambda i: (0, i, 0)),
            _full((64, 128)), _full((1, 64)),
            _full((3, 64)), _full((3, 64)),
        ],
        out_specs=[
            pl.BlockSpec((_NB, 128), lambda i: (i, 0)),
            pl.BlockSpec((_NB, 64), lambda i: (i, 0)),
        ],
        out_shape=[
            jax.ShapeDtypeStruct((_N, 128), jnp.uint32),
            jax.ShapeDtypeStruct((_N, 64), jnp.float32),
        ],
    )(feat2d, xyz, Wm1, cm1, Wa1, Wa2)


def _stage_shortcut(feat2d, Wsc, csc):
    n_blocks = pl.cdiv(_N, _NB)

    def body(feat_ref, wsc_ref, csc_ref, sco_ref):
        X = feat_ref[...].astype(jnp.bfloat16)
        sco = lax.dot_general(wsc_ref[...], X, (((1,), (0,)), ((), ())),
                              preferred_element_type=jnp.float32)
        sco_ref[...] = jnp.maximum(sco + csc_ref[...], 0.0).astype(jnp.bfloat16)

    return pl.pallas_call(
        body,
        grid=(n_blocks,),
        in_specs=[
            pl.BlockSpec((128, _NB), lambda i: (0, i)),
            _full((256, 128)), _full((256, 1)),
        ],
        out_specs=pl.BlockSpec((256, _NB), lambda i: (0, i)),
        out_shape=jax.ShapeDtypeStruct((256, _N), jnp.bfloat16),
    )(feat2d, Wsc, csc)


def _stage_mid(G, V2, W, c):
    n_blocks = pl.cdiv(_N, _NB)

    def body(g_ref, v2_ref, w_ref, c_ref, t2_ref):
        g = _unpack2(g_ref[...]).astype(jnp.bfloat16)
        y = lax.dot_general(g, w_ref[...], (((1,), (1,)), ((), ())),
                            preferred_element_type=jnp.float32)
        f2 = jnp.maximum(y + c_ref[...], 0.0)
        v2 = v2_ref[...]
        t2_ref[:, 0:32] = _pack2(v2[:, 0:32], v2[:, 32:64])
        t2_ref[:, 32:64] = _pack2(f2[:, 0:32], f2[:, 32:64])
        t2_ref[:, 64:128] = jnp.zeros((t2_ref.shape[0], 64), jnp.uint32)

    return pl.pallas_call(
        body,
        grid=(n_blocks,),
        in_specs=[pl.BlockSpec((_NB, 32), lambda i: (i, 0)),
                  pl.BlockSpec((_NB, 64), lambda i: (i, 0)),
                  _full((64, 64)), _full((1, 64))],
        out_specs=pl.BlockSpec((_NB, 128), lambda i: (i, 0)),
        out_shape=jax.ShapeDtypeStruct((_N, 128), jnp.uint32),
    )(G, V2, W, c)


def _stage_out(G2, SCo, Wb2b, cb2b, Wm2, cm2):
    n_blocks = pl.cdiv(_N, _NB)

    def body(g_ref, sco_ref, wb_ref, cb_ref, wm_ref, cm_ref, o_ref):
        g = _unpack2(g_ref[...]).astype(jnp.bfloat16)
        f3 = lax.dot_general(g, wb_ref[...], (((1,), (1,)), ((), ())),
                             preferred_element_type=jnp.float32)
        f3 = jnp.maximum(f3 + cb_ref[...], 0.0).astype(jnp.bfloat16)
        f4 = lax.dot_general(wm_ref[...], f3, (((1,), (1,)), ((), ())),
                             preferred_element_type=jnp.float32)
        f4 = jnp.maximum(f4 + cm_ref[...], 0.0)
        y = f4 + sco_ref[...].astype(jnp.float32)
        o_ref[...] = jnp.maximum(y, 0.2 * y)

    return pl.pallas_call(
        body,
        grid=(n_blocks,),
        in_specs=[pl.BlockSpec((_NB, 32), lambda i: (i, 0)),
                  pl.BlockSpec((256, _NB), lambda i: (0, i)),
                  _full((128, 64)), _full((1, 128)),
                  _full((256, 128)), _full((256, 1))],
        out_specs=pl.BlockSpec((256, _NB), lambda i: (0, i)),
        out_shape=jax.ShapeDtypeStruct((256, _N), jnp.float32),
    )(G2, SCo, Wb2b, cb2b, Wm2, cm2)


def kernel(feature, xyz, neigh_idx,
           W_m1, b_m1, g_m1, be_m1,
           W_b1a, b_b1a, g_b1a, be_b1a,
           W_b1b, b_b1b, g_b1b, be_b1b,
           W_b2a, b_b2a, g_b2a, be_b2a,
           W_b2b, b_b2b, g_b2b, be_b2b,
           W_m2, b_m2, g_m2, be_m2,
           W_sc, b_sc, g_sc, be_sc):
    inv = 1.0 / jnp.sqrt(1.0 + _EPS)

    def scale(W, b, g, be):
        s = g * inv
        return W * s[:, None], (b * s + be)

    We_m1, ce_m1 = scale(W_m1, b_m1, g_m1, be_m1)
    We_b1a, ce_b1a = scale(W_b1a, b_b1a, g_b1a, be_b1a)
    We_b1b, ce_b1b = scale(W_b1b, b_b1b, g_b1b, be_b1b)
    We_b2a, ce_b2a = scale(W_b2a, b_b2a, g_b2a, be_b2a)
    We_b2b, ce_b2b = scale(W_b2b, b_b2b, g_b2b, be_b2b)
    We_m2, ce_m2 = scale(W_m2, b_m2, g_m2, be_m2)
    We_sc, ce_sc = scale(W_sc, b_sc, g_sc, be_sc)

    bf = jnp.bfloat16
    feat2d = feature[0, :, :, 0]
    idxflat = neigh_idx[0].reshape(_N * _K).astype(jnp.int32)
    idx2d = jnp.pad(idxflat, (0, _IDX_ROWS * 128 - _N * _K))

    T1, V2 = _stage_a(
        feat2d, xyz,
        We_m1.astype(bf), ce_m1.reshape(1, 64),
        jnp.transpose(We_b1a), jnp.transpose(We_b2a))
    SCo = _stage_shortcut(feat2d, We_sc.astype(bf), ce_sc.reshape(256, 1))

    G1 = _sc_gather_pool(T1, idx2d, _pack2(ce_b1a[:32], ce_b1a[32:]).reshape(1, 32))
    T2 = _stage_mid(G1, V2, (We_b1b * (1.0 / _K)).astype(bf), ce_b1b.reshape(1, 64))

    G2 = _sc_gather_pool(T2, idx2d, _pack2(ce_b2a[:32], ce_b2a[32:]).reshape(1, 32))
    out = _stage_out(G2, SCo, (We_b2b * (1.0 / _K)).astype(bf), ce_b2b.reshape(1, 128),
                     We_m2.astype(bf), ce_m2.reshape(256, 1))

    return out.reshape(1, 256, _N, 1)
